# Initial kernel scaffold; baseline (speedup 1.0000x reference)
import functools

import jax
import jax.numpy as jnp
from jax import lax
from jax.experimental import pallas as pl
from jax.experimental.pallas import tpu as pltpu
from jax.experimental.pallas import tpu_sc as plsc

_N = 10000
_E = 160000
_D = 256
_H = 128
_KB = 80
_RB = _E // _KB // 16
_KA = 40
_RA = _E // _KA // 32
_NPT = _N // 16
_BI = 2000


def _vsmesh():
    return plsc.VectorSubcoreMesh(core_axis_name="c", subcore_axis_name="s")


@functools.partial(
    pl.kernel,
    out_type=jax.ShapeDtypeStruct((2, _N, 16), jnp.float32),
    mesh=_vsmesh(),
    scratch_types=[
        pltpu.VMEM((_RA, _KA), jnp.int32),
        pltpu.VMEM((_KA, 16), jnp.float32),
        pltpu.VMEM((_RA, 16), jnp.float32),
        pltpu.VMEM_SHARED((_N, 16), jnp.float32),
    ],
)
def _deg_kernel(dst_hbm, out_hbm, dbuf, vbuf, zbuf, hist):
    c = lax.axis_index("c")
    s = lax.axis_index("s")
    zero16 = jnp.zeros((16,), jnp.float32)
    e0 = jnp.where(lax.iota(jnp.int32, 16) == 0, 1.0, 0.0).astype(jnp.float32)

    def fill_z(r, carry):
        zbuf[r, :] = zero16
        return carry

    lax.fori_loop(0, _RA, fill_z, None)

    def fill_v(k, carry):
        vbuf[k, :] = e0
        return carry

    lax.fori_loop(0, _KA, fill_v, None)

    def zero_hist(k, carry):
        pltpu.sync_copy(zbuf, hist.at[pl.ds(s * _NPT + k * _RA, _RA)])
        return carry

    lax.fori_loop(0, _NPT // _RA, zero_hist, None)
    plsc.subcore_barrier()

    wid = c * 16 + s
    pltpu.sync_copy(dst_hbm.at[pl.ds(wid * _RA, _RA)], dbuf)

    def body(j, carry):
        pltpu.sync_copy(vbuf, hist.at[dbuf.at[j]], add=True)
        return carry

    lax.fori_loop(0, _RA, body, None)
    plsc.subcore_barrier()
    pltpu.sync_copy(hist.at[pl.ds(s * _NPT, _NPT)],
                    out_hbm.at[c, pl.ds(s * _NPT, _NPT)])


@functools.partial(
    pl.kernel,
    out_type=jax.ShapeDtypeStruct((2 * _N, _H), jnp.float32),
    mesh=_vsmesh(),
    scratch_types=[
        pltpu.VMEM((_RB, _KB), jnp.int32),
        pltpu.VMEM((_RB, _KB), jnp.int32),
        pltpu.VMEM((_KB, _H), jnp.float32),
        pltpu.VMEM((_RA, _H), jnp.float32),
        pltpu.VMEM_SHARED((_N, _H), jnp.float32),
        pltpu.SemaphoreType.DMA,
    ],
)
def _agg_kernel(y_hbm, src_hbm, dst_hbm, out_hbm, sbuf, dbuf, rowbuf, zbuf,
                acc, sem):
    c = lax.axis_index("c")
    s = lax.axis_index("s")
    zero16 = jnp.zeros((16,), jnp.float32)

    def fill_z(r, carry):
        def fq(q, inner):
            zbuf[r, pl.ds(q * 16, 16)] = zero16
            return inner
        return lax.fori_loop(0, _H // 16, fq, carry)

    lax.fori_loop(0, _RA, fill_z, None)

    def zero_acc(k, carry):
        pltpu.sync_copy(zbuf, acc.at[pl.ds(s * _NPT + k * _RA, _RA)])
        return carry

    lax.fori_loop(0, _NPT // _RA, zero_acc, None)

    pltpu.sync_copy(src_hbm.at[pl.ds(s * _RB, _RB)], sbuf)
    pltpu.sync_copy(dst_hbm.at[pl.ds(s * _RB, _RB)], dbuf)
    base = c * _N

    def offs(j, carry):
        def fq(q, inner):
            sbuf[j, pl.ds(q * 16, 16)] = sbuf[j, pl.ds(q * 16, 16)] + base
            return inner
        return lax.fori_loop(0, _KB // 16, fq, carry)

    lax.fori_loop(0, _RB, offs, None)
    plsc.subcore_barrier()

    def body(j, carry):
        pltpu.async_copy(y_hbm.at[sbuf.at[j]], rowbuf, sem).wait()
        pltpu.sync_copy(rowbuf, acc.at[dbuf.at[j]], add=True)
        return carry

    lax.fori_loop(0, _RB, body, None)
    plsc.subcore_barrier()
    pltpu.sync_copy(acc.at[pl.ds(s * _NPT, _NPT)],
                    out_hbm.at[pl.ds(c * _N + s * _NPT, _NPT)])


def _tc1_body(p0_ref, p1_ref, x_ref, w_ref, y_ref, dinv_ref):
    deg = 1.0 + p0_ref[...] + p1_ref[...]
    dinv = lax.rsqrt(deg)
    xw = jnp.dot(x_ref[...], w_ref[...], preferred_element_type=jnp.float32)
    y_ref[...] = dinv * xw
    dinv_ref[...] = dinv


def _tc1(p0, p1, x, w):
    ni = _N // _BI
    return pl.pallas_call(
        _tc1_body,
        grid=(2, ni),
        in_specs=[
            pl.BlockSpec((_BI, 1), lambda c, i: (i, 0)),
            pl.BlockSpec((_BI, 1), lambda c, i: (i, 0)),
            pl.BlockSpec((_BI, _D), lambda c, i: (i, 0)),
            pl.BlockSpec((_D, _H), lambda c, i: (0, c)),
        ],
        out_specs=[
            pl.BlockSpec((_BI, _H), lambda c, i, _ni=ni: (c * _ni + i, 0)),
            pl.BlockSpec((_BI, 1), lambda c, i: (i, 0)),
        ],
        out_shape=[
            jax.ShapeDtypeStruct((2 * _N, _H), jnp.float32),
            jax.ShapeDtypeStruct((_N, 1), jnp.float32),
        ],
    )(p0, p1, x, w)


def _tc2_body(agg_ref, y_ref, dinv_ref, b_ref, a_ref, w_ref, h1_ref, y2_ref):
    dinv = dinv_ref[...]
    g = agg_ref[...] + y_ref[...]
    h = jnp.concatenate([g[0], g[1]], axis=1)
    h1 = dinv * h + b_ref[...]
    h1_ref[...] = h1
    a = a_ref[0, 0]
    h2 = jnp.where(h1 >= 0, h1, a * h1)
    z = jnp.dot(h2, w_ref[...], preferred_element_type=jnp.float32) * dinv
    y2_ref[0] = z[:, :_H]
    y2_ref[1] = z[:, _H:]


def _tc2(agg1, y1, dinv, b1, a, w2):
    ni = _N // _BI
    return pl.pallas_call(
        _tc2_body,
        grid=(ni,),
        in_specs=[
            pl.BlockSpec((2, _BI, _H), lambda i: (0, i, 0)),
            pl.BlockSpec((2, _BI, _H), lambda i: (0, i, 0)),
            pl.BlockSpec((_BI, 1), lambda i: (i, 0)),
            pl.BlockSpec((1, _D), lambda i: (0, 0)),
            pl.BlockSpec((1, 1), lambda i: (0, 0)),
            pl.BlockSpec((_D, _D), lambda i: (0, 0)),
        ],
        out_specs=[
            pl.BlockSpec((_BI, _D), lambda i: (i, 0)),
            pl.BlockSpec((2, _BI, _H), lambda i: (0, i, 0)),
        ],
        out_shape=[
            jax.ShapeDtypeStruct((_N, _D), jnp.float32),
            jax.ShapeDtypeStruct((2, _N, _H), jnp.float32),
        ],
    )(agg1, y1, dinv, b1, a, w2)


def _tc3_body(agg_ref, y_ref, dinv_ref, b_ref, out_ref):
    g = agg_ref[...] + y_ref[...]
    h = jnp.concatenate([g[0], g[1]], axis=1)
    out_ref[...] = dinv_ref[...] * h + b_ref[...]


def _tc3(agg2, y2, dinv, b2):
    ni = _N // _BI
    return pl.pallas_call(
        _tc3_body,
        grid=(ni,),
        in_specs=[
            pl.BlockSpec((2, _BI, _H), lambda i: (0, i, 0)),
            pl.BlockSpec((2, _BI, _H), lambda i: (0, i, 0)),
            pl.BlockSpec((_BI, 1), lambda i: (i, 0)),
            pl.BlockSpec((1, _D), lambda i: (0, 0)),
        ],
        out_specs=pl.BlockSpec((_BI, _D), lambda i: (i, 0)),
        out_shape=jax.ShapeDtypeStruct((_N, _D), jnp.float32),
    )(agg2, y2, dinv, b2)


def kernel(x, edge_index, W1, b1, prelu_a, W2, b2):
    src = edge_index[0]
    dst = edge_index[1]
    src_r = src.reshape(_E // _KB, _KB)
    dst_r = dst.reshape(_E // _KB, _KB)
    dst_ra = dst.reshape(_E // _KA, _KA)

    parts = _deg_kernel(dst_ra)
    p0 = parts[0, :, :1]
    p1 = parts[1, :, :1]

    y1, dinv = _tc1(p0, p1, x, W1)
    agg1 = _agg_kernel(y1, src_r, dst_r)

    h1, y2 = _tc2(agg1.reshape(2, _N, _H), y1.reshape(2, _N, _H), dinv,
                  b1.reshape(1, _D), prelu_a.reshape(1, 1), W2)

    agg2 = _agg_kernel(y2.reshape(2 * _N, _H), src_r, dst_r)
    out2 = _tc3(agg2.reshape(2, _N, _H), y2, dinv, b2.reshape(1, _D))
    return h1, out2


# speedup vs baseline: 11.6075x; 11.6075x over previous
import functools

import jax
import jax.numpy as jnp
from jax import lax
from jax.experimental import pallas as pl
from jax.experimental.pallas import tpu as pltpu
from jax.experimental.pallas import tpu_sc as plsc

_N = 10000
_E = 160000
_D = 256
_H = 128
_KB = 80
_RB = _E // _KB // 16
_KA = 40
_RA = _E // _KA // 32
_CH = 200
_NCH = _N // _CH
_CPS = (_NCH + 15) // 16
_CHA = _KB
_NCHA = _N // _CHA
_CPSA = (_NCHA + 15) // 16
_BI = 2000


def _vsmesh():
    return plsc.VectorSubcoreMesh(core_axis_name="c", subcore_axis_name="s")


@functools.partial(
    pl.kernel,
    out_type=jax.ShapeDtypeStruct((2, _N, 16), jnp.float32),
    mesh=_vsmesh(),
    scratch_types=[
        pltpu.VMEM((_RA, _KA), jnp.int32),
        pltpu.VMEM((_KA, 16), jnp.float32),
        pltpu.VMEM((_CH, 16), jnp.float32),
        pltpu.VMEM_SHARED((_N, 16), jnp.float32),
    ],
)
def _deg_kernel(dst_hbm, out_hbm, dbuf, vbuf, zbuf, hist):
    c = lax.axis_index("c")
    s = lax.axis_index("s")
    zero16 = jnp.zeros((16,), jnp.float32)
    e0 = jnp.where(lax.iota(jnp.int32, 16) == 0, 1.0, 0.0).astype(jnp.float32)

    def fill_z(r, carry):
        zbuf[r, :] = zero16
        return carry

    lax.fori_loop(0, _CH, fill_z, None)

    def fill_v(k, carry):
        vbuf[k, :] = e0
        return carry

    lax.fori_loop(0, _KA, fill_v, None)

    for k in range(_CPS):
        cid = s + 16 * k

        @pl.when(cid < _NCH)
        def _():
            pltpu.sync_copy(zbuf, hist.at[pl.ds(cid * _CH, _CH)])

    plsc.subcore_barrier()

    wid = c * 16 + s
    pltpu.sync_copy(dst_hbm.at[wid], dbuf)

    def body(j, carry):
        pltpu.sync_copy(vbuf, hist.at[dbuf.at[j]], add=True)
        return carry

    lax.fori_loop(0, _RA, body, None)
    plsc.subcore_barrier()

    for k in range(_CPS):
        cid = s + 16 * k

        @pl.when(cid < _NCH)
        def _():
            pltpu.sync_copy(hist.at[pl.ds(cid * _CH, _CH)],
                            out_hbm.at[c, pl.ds(cid * _CH, _CH)])


@functools.partial(
    pl.kernel,
    out_type=jax.ShapeDtypeStruct((2 * _N, _H), jnp.float32),
    mesh=_vsmesh(),
    scratch_types=[
        pltpu.VMEM((_RB, _KB), jnp.int32),
        pltpu.VMEM((_RB, _KB), jnp.int32),
        pltpu.VMEM((_KB, _H), jnp.float32),
        pltpu.VMEM_SHARED((_N, _H), jnp.float32),
        pltpu.SemaphoreType.DMA,
    ],
)
def _agg_kernel(y_hbm, src_hbm, dst_hbm, out_hbm, sbuf, dbuf, rowbuf,
                acc, sem):
    c = lax.axis_index("c")
    s = lax.axis_index("s")
    zero16 = jnp.zeros((16,), jnp.float32)

    def fill_z(r, carry):
        def fq(q, inner):
            rowbuf[r, pl.ds(q * 16, 16)] = zero16
            return inner
        return lax.fori_loop(0, _H // 16, fq, carry)

    lax.fori_loop(0, _KB, fill_z, None)

    for k in range(_CPSA):
        cid = s + 16 * k

        @pl.when(cid < _NCHA)
        def _():
            pltpu.sync_copy(rowbuf, acc.at[pl.ds(cid * _CHA, _CHA)])

    pltpu.sync_copy(src_hbm.at[s], sbuf)
    pltpu.sync_copy(dst_hbm.at[s], dbuf)
    base = c * _N

    def offs(j, carry):
        def fq(q, inner):
            sbuf[j, pl.ds(q * 16, 16)] = sbuf[j, pl.ds(q * 16, 16)] + base
            return inner
        return lax.fori_loop(0, _KB // 16, fq, carry)

    lax.fori_loop(0, _RB, offs, None)
    plsc.subcore_barrier()

    def body(j, carry):
        pltpu.async_copy(y_hbm.at[sbuf.at[j]], rowbuf, sem).wait()
        pltpu.sync_copy(rowbuf, acc.at[dbuf.at[j]], add=True)
        return carry

    lax.fori_loop(0, _RB, body, None)
    plsc.subcore_barrier()

    for k in range(_CPSA):
        cid = s + 16 * k

        @pl.when(cid < _NCHA)
        def _():
            pltpu.sync_copy(acc.at[pl.ds(cid * _CHA, _CHA)],
                            out_hbm.at[pl.ds(c * _N + cid * _CHA, _CHA)])


def _tc1_body(p0_ref, p1_ref, x_ref, w_ref, y_ref, dinv_ref):
    deg = 1.0 + p0_ref[...] + p1_ref[...]
    dinv = lax.rsqrt(deg)
    xw = jnp.dot(x_ref[...], w_ref[...], preferred_element_type=jnp.float32)
    y_ref[...] = dinv * xw
    dinv_ref[...] = dinv


def _tc1(p0, p1, x, w):
    ni = _N // _BI
    return pl.pallas_call(
        _tc1_body,
        grid=(2, ni),
        in_specs=[
            pl.BlockSpec((_BI, 1), lambda c, i: (i, 0)),
            pl.BlockSpec((_BI, 1), lambda c, i: (i, 0)),
            pl.BlockSpec((_BI, _D), lambda c, i: (i, 0)),
            pl.BlockSpec((_D, _H), lambda c, i: (0, c)),
        ],
        out_specs=[
            pl.BlockSpec((_BI, _H), lambda c, i, _ni=ni: (c * _ni + i, 0)),
            pl.BlockSpec((_BI, 1), lambda c, i: (i, 0)),
        ],
        out_shape=[
            jax.ShapeDtypeStruct((2 * _N, _H), jnp.float32),
            jax.ShapeDtypeStruct((_N, 1), jnp.float32),
        ],
    )(p0, p1, x, w)


def _tc2_body(agg_ref, y_ref, dinv_ref, b_ref, a_ref, w_ref, h1_ref, y2_ref):
    dinv = dinv_ref[...]
    g = agg_ref[...] + y_ref[...]
    h = jnp.concatenate([g[0], g[1]], axis=1)
    h1 = dinv * h + b_ref[...]
    h1_ref[...] = h1
    a = a_ref[0, 0]
    h2 = jnp.where(h1 >= 0, h1, a * h1)
    z = jnp.dot(h2, w_ref[...], preferred_element_type=jnp.float32) * dinv
    y2_ref[0] = z[:, :_H]
    y2_ref[1] = z[:, _H:]


def _tc2(agg1, y1, dinv, b1, a, w2):
    ni = _N // _BI
    return pl.pallas_call(
        _tc2_body,
        grid=(ni,),
        in_specs=[
            pl.BlockSpec((2, _BI, _H), lambda i: (0, i, 0)),
            pl.BlockSpec((2, _BI, _H), lambda i: (0, i, 0)),
            pl.BlockSpec((_BI, 1), lambda i: (i, 0)),
            pl.BlockSpec((1, _D), lambda i: (0, 0)),
            pl.BlockSpec((1, 1), lambda i: (0, 0)),
            pl.BlockSpec((_D, _D), lambda i: (0, 0)),
        ],
        out_specs=[
            pl.BlockSpec((_BI, _D), lambda i: (i, 0)),
            pl.BlockSpec((2, _BI, _H), lambda i: (0, i, 0)),
        ],
        out_shape=[
            jax.ShapeDtypeStruct((_N, _D), jnp.float32),
            jax.ShapeDtypeStruct((2, _N, _H), jnp.float32),
        ],
    )(agg1, y1, dinv, b1, a, w2)


def _tc3_body(agg_ref, y_ref, dinv_ref, b_ref, out_ref):
    g = agg_ref[...] + y_ref[...]
    h = jnp.concatenate([g[0], g[1]], axis=1)
    out_ref[...] = dinv_ref[...] * h + b_ref[...]


def _tc3(agg2, y2, dinv, b2):
    ni = _N // _BI
    return pl.pallas_call(
        _tc3_body,
        grid=(ni,),
        in_specs=[
            pl.BlockSpec((2, _BI, _H), lambda i: (0, i, 0)),
            pl.BlockSpec((2, _BI, _H), lambda i: (0, i, 0)),
            pl.BlockSpec((_BI, 1), lambda i: (i, 0)),
            pl.BlockSpec((1, _D), lambda i: (0, 0)),
        ],
        out_specs=pl.BlockSpec((_BI, _D), lambda i: (i, 0)),
        out_shape=jax.ShapeDtypeStruct((_N, _D), jnp.float32),
    )(agg2, y2, dinv, b2)


def kernel(x, edge_index, W1, b1, prelu_a, W2, b2):
    src = edge_index[0]
    dst = edge_index[1]
    src_r = src.reshape(16, _RB, _KB)
    dst_r = dst.reshape(16, _RB, _KB)
    dst_ra = dst.reshape(32, _RA, _KA)

    parts = _deg_kernel(dst_ra)
    p0 = parts[0, :, :1]
    p1 = parts[1, :, :1]

    y1, dinv = _tc1(p0, p1, x, W1)
    agg1 = _agg_kernel(y1, src_r, dst_r)

    h1, y2 = _tc2(agg1.reshape(2, _N, _H), y1.reshape(2, _N, _H), dinv,
                  b1.reshape(1, _D), prelu_a.reshape(1, 1), W2)

    agg2 = _agg_kernel(y2.reshape(2 * _N, _H), src_r, dst_r)
    out2 = _tc3(agg2.reshape(2, _N, _H), y2, dinv, b2.reshape(1, _D))
    return h1, out2
